# baseline (speedup 1.0000x reference)
import functools

import jax
import jax.numpy as jnp
from jax import lax
from jax.experimental import pallas as pl
from jax.experimental.pallas import tpu as pltpu
from jax.experimental.pallas import tpu_sc as plsc

N = 100000
E = N - 1
H = 128
NCLS = 5

NC = 2
NS = 16
NW = NC * NS

EB = 128
NP = 102400
EP = 102400

_SC_MESH = plsc.VectorSubcoreMesh(core_axis_name="c", subcore_axis_name="s")
_SC_PARAMS = pltpu.CompilerParams(needs_layout_passes=False)


def _zero_fill(buf):
  zeros16 = jnp.zeros((16,), jnp.float32)
  nrows = buf.shape[0]

  def row(r, _):
    for cc in range(8):
      buf[r, pl.ds(cc * 16, 16)] = zeros16
    return 0

  lax.fori_loop(0, nrows, row, 0)


@functools.partial(
    pl.kernel,
    out_type=jax.ShapeDtypeStruct((NP, H), jnp.float32),
    mesh=_SC_MESH,
    scratch_types=[
        pltpu.VMEM((EB,), jnp.int32),
        pltpu.VMEM((EB, H), jnp.float32),
        pltpu.SemaphoreType.DMA,
    ],
    compiler_params=_SC_PARAMS,
)
def _sc_embed_gather(table_hbm, xm_hbm, out_hbm, idx_v, rows_v, sem):
  cid = lax.axis_index("c")
  sid = lax.axis_index("s")
  wid = sid * NC + cid
  rows_per_w = NP // NW
  nwin = rows_per_w // EB

  def win(w, _):
    base = wid * rows_per_w + w * EB
    pltpu.sync_copy(xm_hbm.at[pl.ds(base, EB)], idx_v)
    pltpu.async_copy(table_hbm.at[idx_v], rows_v, sem).wait()
    pltpu.sync_copy(rows_v, out_hbm.at[pl.ds(base, EB)])
    return 0

  lax.fori_loop(0, nwin, win, 0)


ROWS_W = 6400
NPASS = NP // ROWS_W
EPT = EP // NS
TRASH = ROWS_W
FCH = 40
EW = 64
NWIN = EPT // EW
CH = 800


@functools.partial(
    pl.kernel,
    out_type=jax.ShapeDtypeStruct((2 * NP, H), jnp.float32),
    mesh=_SC_MESH,
    scratch_types=[
        pltpu.VMEM((CH,), jnp.int32),
        pltpu.VMEM((CH,), jnp.int32),
        pltpu.VMEM((NWIN + 1, EW), jnp.int32),
        pltpu.VMEM((NWIN + 1, EW), jnp.int32),
        pltpu.VMEM((NWIN + 1, EW), jnp.int32),
        pltpu.VMEM((EW, H), jnp.float32),
        pltpu.VMEM((EW, 2 * H), jnp.float32),
        pltpu.VMEM((EW, H), jnp.float32),
        pltpu.VMEM((FCH, H), jnp.float32),
        pltpu.VMEM_SHARED((ROWS_W + 8, H), jnp.float32),
        pltpu.SemaphoreType.DMA,
    ],
    compiler_params=_SC_PARAMS,
)
def _sc_edge_sweep(h_hbm, huc_hbm, fin_hbm, src_hbm, dst_hbm, out_hbm,
                   cs_b, cd_b, sc_b, dl_b, dg_b,
                   h_v, g_v, f_v, z_v, acc, sem):
  cid = lax.axis_index("c")
  sid = lax.axis_index("s")
  i16 = jnp.int32

  _zero_fill(z_v)

  def zinit(k, _):
    pltpu.sync_copy(z_v,
                    acc.at[pl.ds(sid * (ROWS_W // NS) + k * FCH, FCH)])
    return 0
  lax.fori_loop(0, ROWS_W // NS // FCH, zinit, 0)
  plsc.subcore_barrier()

  zeros_i = jnp.zeros((16,), i16)
  trash_i = jnp.full((16,), TRASH, i16)
  last_l = jnp.full((16,), 15, i16)
  iota16 = lax.iota(i16, 16)

  def one_pass(p, _):
    base = p * ROWS_W
    base_v = jnp.full((16,), base, i16)

    def chunk(q, cnt_v0):
      pltpu.sync_copy(src_hbm.at[pl.ds(sid * EPT + q * CH, CH)], cs_b)
      pltpu.sync_copy(dst_hbm.at[pl.ds(sid * EPT + q * CH, CH)], cd_b)

      def comp(i, cnt_v):
        s16 = cs_b[pl.ds(i * 16, 16)]
        d16 = cd_b[pl.ds(i * 16, 16)]
        m = (d16 >= base_v) & (d16 < base_v + ROWS_W)
        cum = plsc.cumsum(m.astype(i16))
        pos = cnt_v + cum - 1
        pr = lax.shift_right_logical(pos, 6)
        pc = lax.bitwise_and(pos, EW - 1)
        plsc.store_scatter(sc_b, [pr, pc], s16, mask=m)
        plsc.store_scatter(dl_b, [pr, pc], d16 - base_v, mask=m)
        plsc.store_scatter(dg_b, [pr, pc], d16, mask=m)
        return cnt_v + cum.at[last_l].get(mode="promise_in_bounds")

      return lax.fori_loop(0, CH // 16, comp, cnt_v0)

    cnt_v = lax.fori_loop(0, EPT // CH, chunk, jnp.zeros((16,), i16))

    for k in range(EW // 16):
      posp = cnt_v + iota16 + (k * 16)
      ppr = lax.shift_right_logical(posp, 6)
      ppc = lax.bitwise_and(posp, EW - 1)
      plsc.store_scatter(sc_b, [ppr, ppc], zeros_i)
      plsc.store_scatter(dl_b, [ppr, ppc], trash_i)
      plsc.store_scatter(dg_b, [ppr, ppc], zeros_i)

    def wint(t, _):
      live = jnp.any(cnt_v > t * EW)

      @pl.when(live)
      def _wbody():
        _wint_body(t)
      return 0

    def _wint_body(t):
      src_w = sc_b.at[t]
      dl_w = dl_b.at[t]

      @pl.when(cid == 0)
      def _():
        pltpu.async_copy(h_hbm.at[src_w], h_v, sem).wait()
        pltpu.async_copy(h_v, acc.at[dl_w], sem, add=True).wait()

      @pl.when(cid == 1)
      def _():
        cp1 = pltpu.async_copy(huc_hbm.at[src_w], g_v, sem)
        cp3 = pltpu.async_copy(fin_hbm.at[dg_b.at[t]], f_v, sem)
        cp1.wait()
        cp3.wait()

        def row(r, _):
          for cc in range(8):
            sl = pl.ds(cc * 16, 16)
            x = g_v[r, sl] + f_v[r, sl]
            f_v[r, sl] = g_v[r, pl.ds(H + cc * 16, 16)] / (1.0 + jnp.exp(-x))
          return 0

        lax.fori_loop(0, EW, row, 0)
        pltpu.async_copy(f_v, acc.at[dl_w], sem, add=True).wait()

    lax.fori_loop(0, NWIN, wint, 0)
    plsc.subcore_barrier()

    def flush(k, _):
      r = sid * (ROWS_W // NS) + k * FCH
      pltpu.sync_copy(acc.at[pl.ds(r, FCH)],
                      out_hbm.at[pl.ds(cid * NP + base + r, FCH)])
      pltpu.sync_copy(z_v, acc.at[pl.ds(r, FCH)])
      return 0

    lax.fori_loop(0, ROWS_W // NS // FCH, flush, 0)
    plsc.subcore_barrier()
    return 0

  lax.fori_loop(0, NPASS, one_pass, 0)


_BN = 1024
_GRID = NP // _BN


def _row_spec(cols):
  return pl.BlockSpec((_BN, cols), lambda i: (i, 0))


def _full_spec(r, c):
  return pl.BlockSpec((r, c), lambda i: (0, 0))


def _gates(iou):
  i = jax.nn.sigmoid(iou[:, :H])
  o = jax.nn.sigmoid(iou[:, H:2 * H])
  u = jnp.tanh(iou[:, 2 * H:])
  return i, o, u


def _tc_pre_body(emb_ref, mask_ref, wiou_ref, bwiou_ref, wf_ref, bwf_ref,
                 biou_ref, uf_ref,
                 iouin_ref, fin_ref, h_ref, huc_ref):
  e = emb_ref[...]
  m = mask_ref[...][:, 0:1]
  iou_in = (jnp.dot(e, wiou_ref[...], preferred_element_type=jnp.float32)
            + bwiou_ref[...]) * m
  f_in = (jnp.dot(e, wf_ref[...], preferred_element_type=jnp.float32)
          + bwf_ref[...]) * m
  iouin_ref[...] = iou_in
  fin_ref[...] = f_in
  i, o, u = _gates(iou_in + biou_ref[...])
  c = i * u
  h = o * jnp.tanh(c)
  h_ref[...] = h
  hu = jnp.dot(h, uf_ref[...], preferred_element_type=jnp.float32)
  huc_ref[...] = jnp.concatenate([hu, c], axis=1)


_tc_pre = pl.pallas_call(
    _tc_pre_body,
    grid=(_GRID,),
    in_specs=[
        _row_spec(H),
        _row_spec(H),
        _full_spec(H, 3 * H),
        _full_spec(1, 3 * H),
        _full_spec(H, H),
        _full_spec(1, H),
        _full_spec(1, 3 * H),
        _full_spec(H, H),
    ],
    out_specs=[
        _row_spec(3 * H),
        _row_spec(H),
        _row_spec(H),
        _row_spec(2 * H),
    ],
    out_shape=[
        jax.ShapeDtypeStruct((NP, 3 * H), jnp.float32),
        jax.ShapeDtypeStruct((NP, H), jnp.float32),
        jax.ShapeDtypeStruct((NP, H), jnp.float32),
        jax.ShapeDtypeStruct((NP, 2 * H), jnp.float32),
    ],
)


def _tc_step_body(iouin_ref, hsum_ref, cred_ref, uiou_ref, biou_ref, uf_ref,
                  h_ref, huc_ref):
  hs = hsum_ref[...]
  iou = (iouin_ref[...]
         + jnp.dot(hs, uiou_ref[...], preferred_element_type=jnp.float32)
         + biou_ref[...])
  i, o, u = _gates(iou)
  c = i * u + cred_ref[...]
  h = o * jnp.tanh(c)
  h_ref[...] = h
  hu = jnp.dot(h, uf_ref[...], preferred_element_type=jnp.float32)
  huc_ref[...] = jnp.concatenate([hu, c], axis=1)


_tc_step = pl.pallas_call(
    _tc_step_body,
    grid=(_GRID,),
    in_specs=[
        _row_spec(3 * H),
        pl.BlockSpec((_BN, H), lambda i: (i, 0)),
        pl.BlockSpec((_BN, H), lambda i: (i + _GRID, 0)),
        _full_spec(H, 3 * H),
        _full_spec(1, 3 * H),
        _full_spec(H, H),
    ],
    out_specs=[_row_spec(H), _row_spec(2 * H)],
    out_shape=[
        jax.ShapeDtypeStruct((NP, H), jnp.float32),
        jax.ShapeDtypeStruct((NP, 2 * H), jnp.float32),
    ],
)


def _tc_last_body(iouin_ref, hsum_ref, cred_ref, uiou_ref, biou_ref,
                  wout_ref, bout_ref, out_ref):
  hs = hsum_ref[...]
  iou = (iouin_ref[...]
         + jnp.dot(hs, uiou_ref[...], preferred_element_type=jnp.float32)
         + biou_ref[...])
  i, o, u = _gates(iou)
  c = i * u + cred_ref[...]
  h = o * jnp.tanh(c)
  out_ref[...] = (jnp.dot(h, wout_ref[...], preferred_element_type=jnp.float32)
                  + bout_ref[...])


_tc_last = pl.pallas_call(
    _tc_last_body,
    grid=(_GRID,),
    in_specs=[
        _row_spec(3 * H),
        pl.BlockSpec((_BN, H), lambda i: (i, 0)),
        pl.BlockSpec((_BN, H), lambda i: (i + _GRID, 0)),
        _full_spec(H, 3 * H),
        _full_spec(1, 3 * H),
        _full_spec(H, H),
        _full_spec(1, H),
    ],
    out_specs=_row_spec(H),
    out_shape=jax.ShapeDtypeStruct((NP, H), jnp.float32),
)


def kernel(x, mask, edge_index, emb_table, W_iou, b_W_iou, W_f, b_W_f,
           U_iou, b_iou, U_f, W_out, b_out):
  f32 = jnp.float32

  xm = (x * mask).astype(jnp.int32)
  pad_n = NP - N
  xm_p = jnp.concatenate([xm, (jnp.arange(pad_n, dtype=jnp.int32) % 256)])
  maskf = jnp.broadcast_to(mask.astype(f32)[:, None], (N, H))
  maskf_p = jnp.concatenate([maskf, jnp.zeros((pad_n, H), f32)], axis=0)

  src = edge_index[0].astype(jnp.int32)
  dst = edge_index[1].astype(jnp.int32)
  pad_e = EP - E
  pad_idx = N + (jnp.arange(pad_e, dtype=jnp.int32) % 64)
  src_p = jnp.concatenate([src, pad_idx])
  dst_p = jnp.concatenate([dst, pad_idx])

  bwiou_r = b_W_iou.reshape(1, 3 * H)
  bwf_r = b_W_f.reshape(1, H)
  biou_r = b_iou.reshape(1, 3 * H)
  wout_p = jnp.zeros((H, H), f32).at[:, :NCLS].set(W_out)
  bout_p = jnp.zeros((1, H), f32).at[0, :NCLS].set(b_out)

  embeds = _sc_embed_gather(emb_table, xm_p)
  iou_in, f_in, h, huc = _tc_pre(
      embeds, maskf_p, W_iou, bwiou_r, W_f, bwf_r, biou_r, U_f)

  for step in range(2):
    sums = _sc_edge_sweep(h, huc, f_in, src_p, dst_p)
    if step == 0:
      h, huc = _tc_step(iou_in, sums, sums, U_iou, biou_r, U_f)
    else:
      out_full = _tc_last(iou_in, sums, sums, U_iou, biou_r,
                          wout_p, bout_p)

  return out_full[:N, :NCLS]


# speedup vs baseline: 1.3823x; 1.3823x over previous
import functools

import jax
import jax.numpy as jnp
from jax import lax
from jax.experimental import pallas as pl
from jax.experimental.pallas import tpu as pltpu
from jax.experimental.pallas import tpu_sc as plsc

N = 100000
E = N - 1
H = 128
NCLS = 5

NC = 2
NS = 16
NW = NC * NS

EB = 128
NP = 102400
EP = 102400

_SC_MESH = plsc.VectorSubcoreMesh(core_axis_name="c", subcore_axis_name="s")
_SC_PARAMS = pltpu.CompilerParams(needs_layout_passes=False)


def _zero_fill(buf):
  zeros16 = jnp.zeros((16,), jnp.float32)
  nrows = buf.shape[0]

  def row(r, _):
    for cc in range(8):
      buf[r, pl.ds(cc * 16, 16)] = zeros16
    return 0

  lax.fori_loop(0, nrows, row, 0)


@functools.partial(
    pl.kernel,
    out_type=jax.ShapeDtypeStruct((NP, H), jnp.float32),
    mesh=_SC_MESH,
    scratch_types=[
        pltpu.VMEM((EB,), jnp.int32),
        pltpu.VMEM((EB, H), jnp.float32),
        pltpu.SemaphoreType.DMA,
    ],
    compiler_params=_SC_PARAMS,
)
def _sc_embed_gather(table_hbm, xm_hbm, out_hbm, idx_v, rows_v, sem):
  cid = lax.axis_index("c")
  sid = lax.axis_index("s")
  wid = sid * NC + cid
  rows_per_w = NP // NW
  nwin = rows_per_w // EB

  def win(w, _):
    base = wid * rows_per_w + w * EB
    pltpu.sync_copy(xm_hbm.at[pl.ds(base, EB)], idx_v)
    pltpu.async_copy(table_hbm.at[idx_v], rows_v, sem).wait()
    pltpu.sync_copy(rows_v, out_hbm.at[pl.ds(base, EB)])
    return 0

  lax.fori_loop(0, nwin, win, 0)


ROWS_W = 6400
NPASS = NP // ROWS_W
EPT = EP // NS
TRASH = ROWS_W
FCH = 40
EW = 64
NWIN = EPT // EW
CH = 800


@functools.partial(
    pl.kernel,
    out_type=jax.ShapeDtypeStruct((2 * NP, H), jnp.float32),
    mesh=_SC_MESH,
    scratch_types=[
        pltpu.VMEM((CH,), jnp.int32),
        pltpu.VMEM((CH,), jnp.int32),
        pltpu.VMEM((NWIN + 1, EW), jnp.int32),
        pltpu.VMEM((NWIN + 1, EW), jnp.int32),
        pltpu.VMEM((NWIN + 1, EW), jnp.int32),
        pltpu.VMEM((EW, H), jnp.float32),
        pltpu.VMEM((EW, 2 * H), jnp.float32),
        pltpu.VMEM((EW, H), jnp.float32),
        pltpu.VMEM((FCH, H), jnp.float32),
        pltpu.VMEM_SHARED((ROWS_W + 8, H), jnp.float32),
        pltpu.SemaphoreType.DMA,
    ],
    compiler_params=_SC_PARAMS,
)
def _sc_edge_sweep(h_hbm, huc_hbm, fin_hbm, src_hbm, dst_hbm, out_hbm,
                   cs_b, cd_b, sc_b, dl_b, dg_b,
                   h_v, g_v, f_v, z_v, acc, sem):
  cid = lax.axis_index("c")
  sid = lax.axis_index("s")
  i16 = jnp.int32

  _zero_fill(z_v)

  def zinit(k, _):
    pltpu.sync_copy(z_v,
                    acc.at[pl.ds(sid * (ROWS_W // NS) + k * FCH, FCH)])
    return 0
  lax.fori_loop(0, ROWS_W // NS // FCH, zinit, 0)
  plsc.subcore_barrier()

  zeros_i = jnp.zeros((16,), i16)
  trash_i = jnp.full((16,), TRASH, i16)
  last_l = jnp.full((16,), 15, i16)
  iota16 = lax.iota(i16, 16)

  def one_pass(p, _):
    base = p * ROWS_W
    base_v = jnp.full((16,), base, i16)

    def chunk(q, cnt_v0):
      cpa = pltpu.async_copy(src_hbm.at[pl.ds(sid * EPT + q * CH, CH)],
                             cs_b, sem)
      cpb = pltpu.async_copy(dst_hbm.at[pl.ds(sid * EPT + q * CH, CH)],
                             cd_b, sem)
      cpa.wait()
      cpb.wait()

      def comp(i, cnt_v):
        s16 = cs_b[pl.ds(i * 16, 16)]
        d16 = cd_b[pl.ds(i * 16, 16)]
        m = (d16 >= base_v) & (d16 < base_v + ROWS_W)
        cum = plsc.cumsum(m.astype(i16))
        pos = cnt_v + cum - 1
        pr = lax.shift_right_logical(pos, 6)
        pc = lax.bitwise_and(pos, EW - 1)
        plsc.store_scatter(sc_b, [pr, pc], s16, mask=m)
        plsc.store_scatter(dl_b, [pr, pc], d16 - base_v, mask=m)
        plsc.store_scatter(dg_b, [pr, pc], d16, mask=m)
        return cnt_v + cum.at[last_l].get(mode="promise_in_bounds")

      return lax.fori_loop(0, CH // 16, comp, cnt_v0)

    cnt_v = lax.fori_loop(0, EPT // CH, chunk, jnp.zeros((16,), i16))

    for k in range(EW // 16):
      posp = cnt_v + iota16 + (k * 16)
      ppr = lax.shift_right_logical(posp, 6)
      ppc = lax.bitwise_and(posp, EW - 1)
      plsc.store_scatter(sc_b, [ppr, ppc], zeros_i)
      plsc.store_scatter(dl_b, [ppr, ppc], trash_i)
      plsc.store_scatter(dg_b, [ppr, ppc], zeros_i)

    def wint(t, _):
      live = jnp.any(cnt_v > t * EW)

      @pl.when(live)
      def _wbody():
        _wint_body(t)
      return 0

    def _wint_body(t):
      src_w = sc_b.at[t]
      dl_w = dl_b.at[t]

      @pl.when(cid == 0)
      def _():
        pltpu.async_copy(h_hbm.at[src_w], h_v, sem).wait()
        pltpu.async_copy(h_v, acc.at[dl_w], sem, add=True).wait()

      @pl.when(cid == 1)
      def _():
        cp1 = pltpu.async_copy(huc_hbm.at[src_w], g_v, sem)
        cp3 = pltpu.async_copy(fin_hbm.at[dg_b.at[t]], f_v, sem)
        cp1.wait()
        cp3.wait()

        def row(r, _):
          for cc in range(8):
            sl = pl.ds(cc * 16, 16)
            x = g_v[r, sl] + f_v[r, sl]
            f_v[r, sl] = g_v[r, pl.ds(H + cc * 16, 16)] / (1.0 + jnp.exp(-x))
          return 0

        lax.fori_loop(0, EW, row, 0)
        pltpu.async_copy(f_v, acc.at[dl_w], sem, add=True).wait()

    lax.fori_loop(0, NWIN, wint, 0)
    plsc.subcore_barrier()

    r0 = sid * (ROWS_W // NS)
    fcps = [
        pltpu.async_copy(
            acc.at[pl.ds(r0 + k * FCH, FCH)],
            out_hbm.at[pl.ds(cid * NP + base + r0 + k * FCH, FCH)], sem)
        for k in range(ROWS_W // NS // FCH)
    ]
    for cp in fcps:
      cp.wait()
    zcps = [
        pltpu.async_copy(z_v, acc.at[pl.ds(r0 + k * FCH, FCH)], sem)
        for k in range(ROWS_W // NS // FCH)
    ]
    for cp in zcps:
      cp.wait()
    plsc.subcore_barrier()
    return 0

  lax.fori_loop(0, NPASS, one_pass, 0)


_BN = 1024
_GRID = NP // _BN


def _row_spec(cols):
  return pl.BlockSpec((_BN, cols), lambda i: (i, 0))


def _full_spec(r, c):
  return pl.BlockSpec((r, c), lambda i: (0, 0))


def _gates(iou):
  i = jax.nn.sigmoid(iou[:, :H])
  o = jax.nn.sigmoid(iou[:, H:2 * H])
  u = jnp.tanh(iou[:, 2 * H:])
  return i, o, u


def _tc_pre_body(emb_ref, mask_ref, wiou_ref, bwiou_ref, wf_ref, bwf_ref,
                 biou_ref, uf_ref,
                 iouin_ref, fin_ref, h_ref, huc_ref):
  e = emb_ref[...]
  m = mask_ref[...][:, 0:1]
  iou_in = (jnp.dot(e, wiou_ref[...], preferred_element_type=jnp.float32)
            + bwiou_ref[...]) * m
  f_in = (jnp.dot(e, wf_ref[...], preferred_element_type=jnp.float32)
          + bwf_ref[...]) * m
  iouin_ref[...] = iou_in
  fin_ref[...] = f_in
  i, o, u = _gates(iou_in + biou_ref[...])
  c = i * u
  h = o * jnp.tanh(c)
  h_ref[...] = h
  hu = jnp.dot(h, uf_ref[...], preferred_element_type=jnp.float32)
  huc_ref[...] = jnp.concatenate([hu, c], axis=1)


_tc_pre = pl.pallas_call(
    _tc_pre_body,
    grid=(_GRID,),
    in_specs=[
        _row_spec(H),
        _row_spec(H),
        _full_spec(H, 3 * H),
        _full_spec(1, 3 * H),
        _full_spec(H, H),
        _full_spec(1, H),
        _full_spec(1, 3 * H),
        _full_spec(H, H),
    ],
    out_specs=[
        _row_spec(3 * H),
        _row_spec(H),
        _row_spec(H),
        _row_spec(2 * H),
    ],
    out_shape=[
        jax.ShapeDtypeStruct((NP, 3 * H), jnp.float32),
        jax.ShapeDtypeStruct((NP, H), jnp.float32),
        jax.ShapeDtypeStruct((NP, H), jnp.float32),
        jax.ShapeDtypeStruct((NP, 2 * H), jnp.float32),
    ],
)


def _tc_step_body(iouin_ref, hsum_ref, cred_ref, uiou_ref, biou_ref, uf_ref,
                  h_ref, huc_ref):
  hs = hsum_ref[...]
  iou = (iouin_ref[...]
         + jnp.dot(hs, uiou_ref[...], preferred_element_type=jnp.float32)
         + biou_ref[...])
  i, o, u = _gates(iou)
  c = i * u + cred_ref[...]
  h = o * jnp.tanh(c)
  h_ref[...] = h
  hu = jnp.dot(h, uf_ref[...], preferred_element_type=jnp.float32)
  huc_ref[...] = jnp.concatenate([hu, c], axis=1)


_tc_step = pl.pallas_call(
    _tc_step_body,
    grid=(_GRID,),
    in_specs=[
        _row_spec(3 * H),
        pl.BlockSpec((_BN, H), lambda i: (i, 0)),
        pl.BlockSpec((_BN, H), lambda i: (i + _GRID, 0)),
        _full_spec(H, 3 * H),
        _full_spec(1, 3 * H),
        _full_spec(H, H),
    ],
    out_specs=[_row_spec(H), _row_spec(2 * H)],
    out_shape=[
        jax.ShapeDtypeStruct((NP, H), jnp.float32),
        jax.ShapeDtypeStruct((NP, 2 * H), jnp.float32),
    ],
)


def _tc_last_body(iouin_ref, hsum_ref, cred_ref, uiou_ref, biou_ref,
                  wout_ref, bout_ref, out_ref):
  hs = hsum_ref[...]
  iou = (iouin_ref[...]
         + jnp.dot(hs, uiou_ref[...], preferred_element_type=jnp.float32)
         + biou_ref[...])
  i, o, u = _gates(iou)
  c = i * u + cred_ref[...]
  h = o * jnp.tanh(c)
  out_ref[...] = (jnp.dot(h, wout_ref[...], preferred_element_type=jnp.float32)
                  + bout_ref[...])


_tc_last = pl.pallas_call(
    _tc_last_body,
    grid=(_GRID,),
    in_specs=[
        _row_spec(3 * H),
        pl.BlockSpec((_BN, H), lambda i: (i, 0)),
        pl.BlockSpec((_BN, H), lambda i: (i + _GRID, 0)),
        _full_spec(H, 3 * H),
        _full_spec(1, 3 * H),
        _full_spec(H, H),
        _full_spec(1, H),
    ],
    out_specs=_row_spec(H),
    out_shape=jax.ShapeDtypeStruct((NP, H), jnp.float32),
)


def kernel(x, mask, edge_index, emb_table, W_iou, b_W_iou, W_f, b_W_f,
           U_iou, b_iou, U_f, W_out, b_out):
  f32 = jnp.float32

  xm = x.astype(jnp.int32)
  pad_n = NP - N
  xm_p = jnp.concatenate([xm, (jnp.arange(pad_n, dtype=jnp.int32) % 256)])
  maskf = jnp.broadcast_to(mask.astype(f32)[:, None], (N, H))
  maskf_p = jnp.concatenate([maskf, jnp.zeros((pad_n, H), f32)], axis=0)

  src = edge_index[0].astype(jnp.int32)
  dst = edge_index[1].astype(jnp.int32)
  pad_e = EP - E
  pad_idx = N + (jnp.arange(pad_e, dtype=jnp.int32) % 64)
  src_p = jnp.concatenate([src, pad_idx])
  dst_p = jnp.concatenate([dst, pad_idx])

  bwiou_r = b_W_iou.reshape(1, 3 * H)
  bwf_r = b_W_f.reshape(1, H)
  biou_r = b_iou.reshape(1, 3 * H)
  wout_p = jnp.zeros((H, H), f32).at[:, :NCLS].set(W_out)
  bout_p = jnp.zeros((1, H), f32).at[0, :NCLS].set(b_out)

  embeds = _sc_embed_gather(emb_table, xm_p)
  iou_in, f_in, h, huc = _tc_pre(
      embeds, maskf_p, W_iou, bwiou_r, W_f, bwf_r, biou_r, U_f)

  for step in range(2):
    sums = _sc_edge_sweep(h, huc, f_in, src_p, dst_p)
    if step == 0:
      h, huc = _tc_step(iou_in, sums, sums, U_iou, biou_r, U_f)
    else:
      out_full = _tc_last(iou_in, sums, sums, U_iou, biou_r,
                          wout_p, bout_p)

  return out_full[:N, :NCLS]


# speedup vs baseline: 1.7685x; 1.2794x over previous
import functools

import jax
import jax.numpy as jnp
from jax import lax
from jax.experimental import pallas as pl
from jax.experimental.pallas import tpu as pltpu
from jax.experimental.pallas import tpu_sc as plsc

N = 100000
E = N - 1
H = 128
NCLS = 5

NC = 2
NS = 16
NW = NC * NS

EB = 128
NP = 102400
EP = 102400

_SC_MESH = plsc.VectorSubcoreMesh(core_axis_name="c", subcore_axis_name="s")
_SC_PARAMS = pltpu.CompilerParams(needs_layout_passes=False)


def _zero_fill(buf):
  zeros16 = jnp.zeros((16,), jnp.float32)
  nrows = buf.shape[0]

  def row(r, _):
    for cc in range(8):
      buf[r, pl.ds(cc * 16, 16)] = zeros16
    return 0

  lax.fori_loop(0, nrows, row, 0)


@functools.partial(
    pl.kernel,
    out_type=jax.ShapeDtypeStruct((NP, H), jnp.float32),
    mesh=_SC_MESH,
    scratch_types=[
        pltpu.VMEM((EB,), jnp.int32),
        pltpu.VMEM((EB, H), jnp.float32),
        pltpu.SemaphoreType.DMA,
    ],
    compiler_params=_SC_PARAMS,
)
def _sc_embed_gather(table_hbm, xm_hbm, out_hbm, idx_v, rows_v, sem):
  cid = lax.axis_index("c")
  sid = lax.axis_index("s")
  wid = sid * NC + cid
  rows_per_w = NP // NW
  nwin = rows_per_w // EB

  def win(w, _):
    base = wid * rows_per_w + w * EB
    pltpu.sync_copy(xm_hbm.at[pl.ds(base, EB)], idx_v)
    pltpu.async_copy(table_hbm.at[idx_v], rows_v, sem).wait()
    pltpu.sync_copy(rows_v, out_hbm.at[pl.ds(base, EB)])
    return 0

  lax.fori_loop(0, nwin, win, 0)


ROWS_W = 6400
NPASS = NP // ROWS_W
EPT = EP // NS
TRASH = ROWS_W
FCH = 40
EW = 64
NWIN = EPT // EW
CH = 800


@functools.partial(
    pl.kernel,
    out_type=jax.ShapeDtypeStruct((2 * NP, H), jnp.float32),
    mesh=_SC_MESH,
    scratch_types=[
        pltpu.VMEM((CH,), jnp.int32),
        pltpu.VMEM((CH,), jnp.int32),
        pltpu.VMEM((NWIN + 1, EW), jnp.int32),
        pltpu.VMEM((NWIN + 1, EW), jnp.int32),
        pltpu.VMEM((NWIN + 1, EW), jnp.int32),
        pltpu.VMEM((EW, H), jnp.float32),
        pltpu.VMEM((EW, 2 * H), jnp.float32),
        pltpu.VMEM((EW, H), jnp.float32),
        pltpu.VMEM((FCH, H), jnp.float32),
        pltpu.VMEM_SHARED((ROWS_W + 8, H), jnp.float32),
        pltpu.SemaphoreType.DMA,
    ],
    compiler_params=_SC_PARAMS,
)
def _sc_edge_sweep(h_hbm, huc_hbm, fin_hbm, src_hbm, dst_hbm, out_hbm,
                   cs_b, cd_b, sc_b, dl_b, dg_b,
                   h_v, g_v, f_v, z_v, acc, sem):
  cid = lax.axis_index("c")
  sid = lax.axis_index("s")
  i16 = jnp.int32

  _zero_fill(z_v)

  def zinit(k, _):
    pltpu.sync_copy(z_v,
                    acc.at[pl.ds(sid * (ROWS_W // NS) + k * FCH, FCH)])
    return 0
  lax.fori_loop(0, ROWS_W // NS // FCH, zinit, 0)
  plsc.subcore_barrier()

  zeros_i = jnp.zeros((16,), i16)
  trash_i = jnp.full((16,), TRASH, i16)
  last_l = jnp.full((16,), 15, i16)
  iota16 = lax.iota(i16, 16)

  def one_pass(p, _):
    base = p * ROWS_W
    base_v = jnp.full((16,), base, i16)
    ph = lax.rem(p + cid, 2)

    def chunk(q, cnt_v0):
      cpa = pltpu.async_copy(src_hbm.at[pl.ds(sid * EPT + q * CH, CH)],
                             cs_b, sem)
      cpb = pltpu.async_copy(dst_hbm.at[pl.ds(sid * EPT + q * CH, CH)],
                             cd_b, sem)
      cpa.wait()
      cpb.wait()

      def comp(i, cnt_v):
        s16 = cs_b[pl.ds(i * 16, 16)]
        d16 = cd_b[pl.ds(i * 16, 16)]
        m = (d16 >= base_v) & (d16 < base_v + ROWS_W)
        cum = plsc.cumsum(m.astype(i16))
        pos = cnt_v + cum - 1
        pr = lax.shift_right_logical(pos, 6)
        pc = lax.bitwise_and(pos, EW - 1)
        plsc.store_scatter(sc_b, [pr, pc], s16, mask=m)
        plsc.store_scatter(dl_b, [pr, pc], d16 - base_v, mask=m)
        plsc.store_scatter(dg_b, [pr, pc], d16, mask=m)
        return cnt_v + cum.at[last_l].get(mode="promise_in_bounds")

      return lax.fori_loop(0, CH // 16, comp, cnt_v0)

    cnt_v = lax.fori_loop(0, EPT // CH, chunk, jnp.zeros((16,), i16))

    for k in range(EW // 16):
      posp = cnt_v + iota16 + (k * 16)
      ppr = lax.shift_right_logical(posp, 6)
      ppc = lax.bitwise_and(posp, EW - 1)
      plsc.store_scatter(sc_b, [ppr, ppc], zeros_i)
      plsc.store_scatter(dl_b, [ppr, ppc], trash_i)
      plsc.store_scatter(dg_b, [ppr, ppc], zeros_i)

    def wint(t, _):
      live = jnp.any(cnt_v > t * EW)

      @pl.when(live)
      def _wbody():
        _wint_body(t)
      return 0

    def _wint_body(t):
      src_w = sc_b.at[t]
      dl_w = dl_b.at[t]

      @pl.when(ph == 0)
      def _():
        pltpu.async_copy(h_hbm.at[src_w], h_v, sem).wait()
        pltpu.async_copy(h_v, acc.at[dl_w], sem, add=True).wait()

      @pl.when(ph == 1)
      def _():
        cp1 = pltpu.async_copy(huc_hbm.at[src_w], g_v, sem)
        cp3 = pltpu.async_copy(fin_hbm.at[dg_b.at[t]], f_v, sem)
        cp1.wait()
        cp3.wait()

        def row(r, _):
          for cc in range(8):
            sl = pl.ds(cc * 16, 16)
            x = g_v[r, sl] + f_v[r, sl]
            f_v[r, sl] = g_v[r, pl.ds(H + cc * 16, 16)] / (1.0 + jnp.exp(-x))
          return 0

        lax.fori_loop(0, EW, row, 0)
        pltpu.async_copy(f_v, acc.at[dl_w], sem, add=True).wait()

    lax.fori_loop(0, NWIN, wint, 0)
    plsc.subcore_barrier()

    r0 = sid * (ROWS_W // NS)
    fcps = [
        pltpu.async_copy(
            acc.at[pl.ds(r0 + k * FCH, FCH)],
            out_hbm.at[pl.ds(ph * NP + base + r0 + k * FCH, FCH)], sem)
        for k in range(ROWS_W // NS // FCH)
    ]
    for cp in fcps:
      cp.wait()
    zcps = [
        pltpu.async_copy(z_v, acc.at[pl.ds(r0 + k * FCH, FCH)], sem)
        for k in range(ROWS_W // NS // FCH)
    ]
    for cp in zcps:
      cp.wait()
    plsc.subcore_barrier()
    return 0

  lax.fori_loop(0, NPASS, one_pass, 0)


_BN = 1024
_GRID = NP // _BN


def _row_spec(cols):
  return pl.BlockSpec((_BN, cols), lambda i: (i, 0))


def _full_spec(r, c):
  return pl.BlockSpec((r, c), lambda i: (0, 0))


def _gates(iou):
  i = jax.nn.sigmoid(iou[:, :H])
  o = jax.nn.sigmoid(iou[:, H:2 * H])
  u = jnp.tanh(iou[:, 2 * H:])
  return i, o, u


def _tc_pre_body(emb_ref, mask_ref, wiou_ref, bwiou_ref, wf_ref, bwf_ref,
                 biou_ref, uf_ref,
                 iouin_ref, fin_ref, h_ref, huc_ref):
  e = emb_ref[...]
  m = mask_ref[...][:, 0:1]
  iou_in = (jnp.dot(e, wiou_ref[...], preferred_element_type=jnp.float32)
            + bwiou_ref[...]) * m
  f_in = (jnp.dot(e, wf_ref[...], preferred_element_type=jnp.float32)
          + bwf_ref[...]) * m
  iouin_ref[...] = iou_in
  fin_ref[...] = f_in
  i, o, u = _gates(iou_in + biou_ref[...])
  c = i * u
  h = o * jnp.tanh(c)
  h_ref[...] = h
  hu = jnp.dot(h, uf_ref[...], preferred_element_type=jnp.float32)
  huc_ref[...] = jnp.concatenate([hu, c], axis=1)


_tc_pre = pl.pallas_call(
    _tc_pre_body,
    grid=(_GRID,),
    in_specs=[
        _row_spec(H),
        _row_spec(H),
        _full_spec(H, 3 * H),
        _full_spec(1, 3 * H),
        _full_spec(H, H),
        _full_spec(1, H),
        _full_spec(1, 3 * H),
        _full_spec(H, H),
    ],
    out_specs=[
        _row_spec(3 * H),
        _row_spec(H),
        _row_spec(H),
        _row_spec(2 * H),
    ],
    out_shape=[
        jax.ShapeDtypeStruct((NP, 3 * H), jnp.float32),
        jax.ShapeDtypeStruct((NP, H), jnp.float32),
        jax.ShapeDtypeStruct((NP, H), jnp.float32),
        jax.ShapeDtypeStruct((NP, 2 * H), jnp.float32),
    ],
)


def _tc_step_body(iouin_ref, hsum_ref, cred_ref, uiou_ref, biou_ref, uf_ref,
                  h_ref, huc_ref):
  hs = hsum_ref[...]
  iou = (iouin_ref[...]
         + jnp.dot(hs, uiou_ref[...], preferred_element_type=jnp.float32)
         + biou_ref[...])
  i, o, u = _gates(iou)
  c = i * u + cred_ref[...]
  h = o * jnp.tanh(c)
  h_ref[...] = h
  hu = jnp.dot(h, uf_ref[...], preferred_element_type=jnp.float32)
  huc_ref[...] = jnp.concatenate([hu, c], axis=1)


_tc_step = pl.pallas_call(
    _tc_step_body,
    grid=(_GRID,),
    in_specs=[
        _row_spec(3 * H),
        pl.BlockSpec((_BN, H), lambda i: (i, 0)),
        pl.BlockSpec((_BN, H), lambda i: (i + _GRID, 0)),
        _full_spec(H, 3 * H),
        _full_spec(1, 3 * H),
        _full_spec(H, H),
    ],
    out_specs=[_row_spec(H), _row_spec(2 * H)],
    out_shape=[
        jax.ShapeDtypeStruct((NP, H), jnp.float32),
        jax.ShapeDtypeStruct((NP, 2 * H), jnp.float32),
    ],
)


def _tc_last_body(iouin_ref, hsum_ref, cred_ref, uiou_ref, biou_ref,
                  wout_ref, bout_ref, out_ref):
  hs = hsum_ref[...]
  iou = (iouin_ref[...]
         + jnp.dot(hs, uiou_ref[...], preferred_element_type=jnp.float32)
         + biou_ref[...])
  i, o, u = _gates(iou)
  c = i * u + cred_ref[...]
  h = o * jnp.tanh(c)
  out_ref[...] = (jnp.dot(h, wout_ref[...], preferred_element_type=jnp.float32)
                  + bout_ref[...])


_tc_last = pl.pallas_call(
    _tc_last_body,
    grid=(_GRID,),
    in_specs=[
        _row_spec(3 * H),
        pl.BlockSpec((_BN, H), lambda i: (i, 0)),
        pl.BlockSpec((_BN, H), lambda i: (i + _GRID, 0)),
        _full_spec(H, 3 * H),
        _full_spec(1, 3 * H),
        _full_spec(H, H),
        _full_spec(1, H),
    ],
    out_specs=_row_spec(H),
    out_shape=jax.ShapeDtypeStruct((NP, H), jnp.float32),
)


def kernel(x, mask, edge_index, emb_table, W_iou, b_W_iou, W_f, b_W_f,
           U_iou, b_iou, U_f, W_out, b_out):
  f32 = jnp.float32

  xm = x.astype(jnp.int32)
  pad_n = NP - N
  xm_p = jnp.concatenate([xm, (jnp.arange(pad_n, dtype=jnp.int32) % 256)])
  maskf = jnp.broadcast_to(mask.astype(f32)[:, None], (N, H))
  maskf_p = jnp.concatenate([maskf, jnp.zeros((pad_n, H), f32)], axis=0)

  src = edge_index[0].astype(jnp.int32)
  dst = edge_index[1].astype(jnp.int32)
  pad_e = EP - E
  pad_idx = N + (jnp.arange(pad_e, dtype=jnp.int32) % 64)
  src_p = jnp.concatenate([src, pad_idx])
  dst_p = jnp.concatenate([dst, pad_idx])

  bwiou_r = b_W_iou.reshape(1, 3 * H)
  bwf_r = b_W_f.reshape(1, H)
  biou_r = b_iou.reshape(1, 3 * H)
  wout_p = jnp.zeros((H, H), f32).at[:, :NCLS].set(W_out)
  bout_p = jnp.zeros((1, H), f32).at[0, :NCLS].set(b_out)

  embeds = _sc_embed_gather(emb_table, xm_p)
  iou_in, f_in, h, huc = _tc_pre(
      embeds, maskf_p, W_iou, bwiou_r, W_f, bwf_r, biou_r, U_f)

  for step in range(2):
    sums = _sc_edge_sweep(h, huc, f_in, src_p, dst_p)
    if step == 0:
      h, huc = _tc_step(iou_in, sums, sums, U_iou, biou_r, U_f)
    else:
      out_full = _tc_last(iou_in, sums, sums, U_iou, biou_r,
                          wout_p, bout_p)

  return out_full[:N, :NCLS]
